# baseline (speedup 1.0000x reference)
import functools

import jax
import jax.numpy as jnp
from jax import lax
from jax.experimental import pallas as pl
from jax.experimental.pallas import tpu as pltpu
from jax.experimental.pallas import tpu_sc as plsc

B = 4
S = 2048
D = 768
L = 16
NC = 2
NS = 16
NW = NC * NS
POS_W = S // NW
CP = 16
CHUNKS = POS_W // CP
ROWS = B * CP
DV = D // L

def _emb_lookup_body(xt_hbm, tab_hbm, pos_hbm, out_hbm, idx_v, rows_v, pos_v, sem):
    wid = lax.axis_index("s") * NC + lax.axis_index("c")
    p_w = wid * POS_W
    pltpu.sync_copy(xt_hbm.at[wid], idx_v)
    for c in range(CHUNKS):
        p0 = p_w + c * CP
        pltpu.sync_copy(pos_hbm.at[pl.ds(p0, CP)], pos_v)
        pltpu.async_copy(tab_hbm.at[idx_v.at[c]], rows_v, sem).wait()

        def i_body(i, carry):
            def d_body(d, carry2):
                pv = pos_v[i, pl.ds(d * L, L)]
                for b in range(B):
                    plsc.addupdate(rows_v.at[b * CP + i, pl.ds(d * L, L)], pv)
                return carry2

            return lax.fori_loop(0, DV, d_body, carry)

        lax.fori_loop(0, CP, i_body, 0)

        for b in range(B):
            pltpu.sync_copy(
                rows_v.at[pl.ds(b * CP, CP)],
                out_hbm.at[pl.ds(b * S + p0, CP)],
            )


@functools.cache
def _build(interpret: bool = False):
    mesh = plsc.VectorSubcoreMesh(
        core_axis_name="c", subcore_axis_name="s", num_cores=NC, num_subcores=NS
    )
    return pl.kernel(
        _emb_lookup_body,
        out_type=jax.ShapeDtypeStruct((B * S, D), jnp.float32),
        mesh=mesh,
        scratch_types=[
            pltpu.VMEM((CHUNKS, ROWS), jnp.int32),
            pltpu.VMEM((ROWS, D), jnp.float32),
            pltpu.VMEM((CP, D), jnp.float32),
            pltpu.SemaphoreType.DMA,
        ],
        interpret=interpret,
    )


def kernel(x, input_embeddings, positional_embeddings):
    xt = (
        x.astype(jnp.int32)
        .reshape(B, NW, CHUNKS, CP)
        .transpose(1, 2, 0, 3)
        .reshape(NW, CHUNKS, ROWS)
    )
    out = _build()(xt, input_embeddings, positional_embeddings)
    return out.reshape(B, S, D)


# speedup vs baseline: 1.2494x; 1.2494x over previous
import functools

import jax
import jax.numpy as jnp
from jax import lax
from jax.experimental import pallas as pl
from jax.experimental.pallas import tpu as pltpu
from jax.experimental.pallas import tpu_sc as plsc

B = 4
S = 2048
D = 768
L = 16
NC = 2
NS = 16
NW = NC * NS
POS_W = S // NW
CP = 16
CHUNKS = POS_W // CP
ROWS = B * CP
DV = D // L

def _emb_lookup_body(
    xt_hbm, tab_hbm, pos_hbm, out_hbm,
    idx_v, rows_v, pos_v, sg0, sg1, sp0, sp1, so0, so1,
):
    wid = lax.axis_index("s") * NC + lax.axis_index("c")
    p_w = wid * POS_W
    pltpu.sync_copy(xt_hbm.at[wid], idx_v)
    sg = (sg0, sg1)
    sp = (sp0, sp1)
    so = (so0, so1)

    def start_in(c):
        slot = c % 2
        g = pltpu.async_copy(tab_hbm.at[idx_v.at[c]], rows_v.at[slot], sg[slot])
        p = pltpu.async_copy(
            pos_hbm.at[pl.ds(p_w + c * CP, CP)], pos_v.at[slot], sp[slot]
        )
        return g, p

    def start_out(c):
        slot = c % 2
        return [
            pltpu.async_copy(
                rows_v.at[slot, pl.ds(b * CP, CP)],
                out_hbm.at[pl.ds(b * S + p_w + c * CP, CP)],
                so[slot],
            )
            for b in range(B)
        ]

    in_d = {0: start_in(0)}
    out_d = {}
    for c in range(CHUNKS):
        slot = c % 2
        if c + 1 < CHUNKS:
            if c - 1 >= 0:
                for dsc in out_d[c - 1]:
                    dsc.wait()
            in_d[c + 1] = start_in(c + 1)
        g, p = in_d[c]
        g.wait()
        p.wait()

        def i_body(i, carry):
            def d_body(d, carry2):
                pv = pos_v[slot, i, pl.ds(d * L, L)]
                for b in range(B):
                    plsc.addupdate(
                        rows_v.at[slot, b * CP + i, pl.ds(d * L, L)], pv
                    )
                return carry2

            return lax.fori_loop(0, DV, d_body, carry, unroll=4)

        lax.fori_loop(0, CP, i_body, 0)
        out_d[c] = start_out(c)

    for c in (CHUNKS - 2, CHUNKS - 1):
        for dsc in out_d[c]:
            dsc.wait()


@functools.cache
def _build(interpret: bool = False):
    mesh = plsc.VectorSubcoreMesh(
        core_axis_name="c", subcore_axis_name="s", num_cores=NC, num_subcores=NS
    )
    return pl.kernel(
        _emb_lookup_body,
        out_type=jax.ShapeDtypeStruct((B * S, D), jnp.float32),
        mesh=mesh,
        scratch_types=[
            pltpu.VMEM((CHUNKS, ROWS), jnp.int32),
            pltpu.VMEM((2, ROWS, D), jnp.float32),
            pltpu.VMEM((2, CP, D), jnp.float32),
            pltpu.SemaphoreType.DMA,
            pltpu.SemaphoreType.DMA,
            pltpu.SemaphoreType.DMA,
            pltpu.SemaphoreType.DMA,
            pltpu.SemaphoreType.DMA,
            pltpu.SemaphoreType.DMA,
        ],
        interpret=interpret,
    )


def kernel(x, input_embeddings, positional_embeddings):
    xt = (
        x.astype(jnp.int32)
        .reshape(B, NW, CHUNKS, CP)
        .transpose(1, 2, 0, 3)
        .reshape(NW, CHUNKS, ROWS)
    )
    out = _build()(xt, input_embeddings, positional_embeddings)
    return out.reshape(B, S, D)


# speedup vs baseline: 1.2518x; 1.0019x over previous
import functools

import jax
import jax.numpy as jnp
from jax import lax
from jax.experimental import pallas as pl
from jax.experimental.pallas import tpu as pltpu
from jax.experimental.pallas import tpu_sc as plsc

B = 4
S = 2048
D = 768
L = 16
NC = 2
NS = 16
NW = NC * NS
POS_W = S // NW
CP = 8
CHUNKS = POS_W // CP
ROWS = B * CP
DV = D // L
NSLOT = 3


def _emb_lookup_body(xt_hbm, tab_hbm, pos_hbm, out_hbm, idx_v, rows_v, pos_v, *sems):
    sg = sems[0:NSLOT]
    sp = sems[NSLOT:2 * NSLOT]
    so = sems[2 * NSLOT:3 * NSLOT]
    wid = lax.axis_index("s") * NC + lax.axis_index("c")
    p_w = wid * POS_W
    pltpu.sync_copy(xt_hbm.at[wid], idx_v)

    def start_in(c):
        slot = c % NSLOT
        g = pltpu.async_copy(tab_hbm.at[idx_v.at[c]], rows_v.at[slot], sg[slot])
        p = pltpu.async_copy(
            pos_hbm.at[pl.ds(p_w + c * CP, CP)], pos_v.at[slot], sp[slot]
        )
        return g, p

    def start_out(c):
        slot = c % NSLOT
        return [
            pltpu.async_copy(
                rows_v.at[slot, pl.ds(b * CP, CP)],
                out_hbm.at[pl.ds(b * S + p_w + c * CP, CP)],
                so[slot],
            )
            for b in range(B)
        ]

    in_d = {0: start_in(0), 1: start_in(1)}
    out_d = {}
    for c in range(CHUNKS):
        slot = c % NSLOT
        nxt = c + 2
        if nxt < CHUNKS:
            if nxt - NSLOT >= 0:
                for dsc in out_d[nxt - NSLOT]:
                    dsc.wait()
            in_d[nxt] = start_in(nxt)
        g, p = in_d[c]
        g.wait()
        p.wait()

        def i_body(i, carry):
            def d_body(d, carry2):
                pv = pos_v[slot, i, pl.ds(d * L, L)]
                for b in range(B):
                    plsc.addupdate(
                        rows_v.at[slot, b * CP + i, pl.ds(d * L, L)], pv
                    )
                return carry2

            return lax.fori_loop(0, DV, d_body, carry, unroll=4)

        lax.fori_loop(0, CP, i_body, 0)
        out_d[c] = start_out(c)

    for c in range(max(0, CHUNKS - NSLOT), CHUNKS):
        for dsc in out_d[c]:
            dsc.wait()


@functools.cache
def _build(interpret: bool = False):
    mesh = plsc.VectorSubcoreMesh(
        core_axis_name="c", subcore_axis_name="s", num_cores=NC, num_subcores=NS
    )
    sems = [pltpu.SemaphoreType.DMA] * (3 * NSLOT)
    return pl.kernel(
        _emb_lookup_body,
        out_type=jax.ShapeDtypeStruct((B * S, D), jnp.float32),
        mesh=mesh,
        scratch_types=[
            pltpu.VMEM((CHUNKS, ROWS), jnp.int32),
            pltpu.VMEM((NSLOT, ROWS, D), jnp.float32),
            pltpu.VMEM((NSLOT, CP, D), jnp.float32),
        ] + sems,
        interpret=interpret,
    )


def kernel(x, input_embeddings, positional_embeddings):
    xt = (
        x.astype(jnp.int32)
        .reshape(B, NW, CHUNKS, CP)
        .transpose(1, 2, 0, 3)
        .reshape(NW, CHUNKS, ROWS)
    )
    out = _build()(xt, input_embeddings, positional_embeddings)
    return out.reshape(B, S, D)
